# baseline (speedup 1.0000x reference)
import functools

import jax
import jax.numpy as jnp
from jax import lax
from jax.experimental import pallas as pl
from jax.experimental.pallas import tpu as pltpu
from jax.experimental.pallas import tpu_sc as plsc

EPS = 1e-06
N_BINS = 256
_B, _T, _F = 256, 20, 32
_L = 16
_NC, _NS = 2, 16
_NW = _NC * _NS
_N_ROWS = _B * _T * _F
_RPW = _N_ROWS // _NW
_BPW = _RPW // (_T * _F)
_R = 160
_TG = _R // _F
_G = _RPW // _R
_GPB = (_T * _F) // _R


def _sc_body(x_hbm, out_hbm, xin_v, buf_v, pidx_v, sem0, sem1):
    c = lax.axis_index("c")
    s = lax.axis_index("s")
    wid = s * _NC + c
    base = wid * _RPW
    pltpu.sync_copy(x_hbm.at[pl.ds(base, _RPW)], xin_v)

    iota = lax.iota(jnp.int32, _L)
    ones = jnp.full((_L,), 1.0, jnp.float32)
    zeros = jnp.zeros((_L,), jnp.float32)

    def zrow(i, carry):
        t = i // _F
        f = i % _F
        for ch in range(N_BINS // _L):
            buf_v[t, f, pl.ds(ch * _L, _L)] = zeros
        return carry

    lax.fori_loop(0, 2 * _TG * _F, zrow, 0)

    sems = (sem0, sem1)

    def outer(go, carry):
        for b in range(2):
            g = go * 2 + b
            sem = sems[b]

            @pl.when(g >= 2)
            def _wait_and_scrub():
                pltpu.make_async_copy(
                    buf_v.at[pl.ds(b * _TG, _TG)],
                    out_hbm.at[0, pl.ds(0, _TG)],
                    sem,
                ).wait()

                def scrub(j, inner):
                    r = j * _L + iota
                    cols = pidx_v[pl.ds(b * _R + j * _L, _L)]
                    plsc.store_scatter(
                        buf_v,
                        [b * _TG + (r >> 5), r & (_F - 1), cols],
                        zeros)
                    return inner

                lax.fori_loop(0, _R // _L, scrub, 0)

            def setone(j, inner):
                xv = xin_v[pl.ds(g * _R + j * _L, _L)]
                xc = jnp.clip(xv, EPS, 1.0 - EPS)
                idx = jnp.clip((xc * N_BINS).astype(jnp.int32), 0, N_BINS - 1)
                pidx_v[pl.ds(b * _R + j * _L, _L)] = idx
                r = j * _L + iota
                plsc.store_scatter(
                    buf_v,
                    [b * _TG + (r >> 5), r & (_F - 1), idx],
                    ones)
                return inner

            lax.fori_loop(0, _R // _L, setone, 0)
            bb = wid * _BPW + g // _GPB
            t0 = (g % _GPB) * _TG
            pltpu.async_copy(
                buf_v.at[pl.ds(b * _TG, _TG)],
                out_hbm.at[bb, pl.ds(t0, _TG)],
                sem,
            )
        return carry

    lax.fori_loop(0, _G // 2, outer, 0)

    for b in range(2):
        pltpu.make_async_copy(
            buf_v.at[pl.ds(b * _TG, _TG)],
            out_hbm.at[0, pl.ds(0, _TG)],
            sems[b],
        ).wait()


_sc_onehot = functools.partial(
    pl.kernel,
    out_type=jax.ShapeDtypeStruct((_B, _T, _F, N_BINS), jnp.float32),
    mesh=plsc.VectorSubcoreMesh(
        core_axis_name="c", subcore_axis_name="s",
        num_cores=_NC, num_subcores=_NS,
    ),
    scratch_types=[
        pltpu.VMEM((_RPW,), jnp.float32),
        pltpu.VMEM((2 * _TG, _F, N_BINS), jnp.float32),
        pltpu.VMEM((2 * _R,), jnp.int32),
        pltpu.SemaphoreType.DMA,
        pltpu.SemaphoreType.DMA,
    ],
    compiler_params=pltpu.CompilerParams(
        use_tc_tiling_on_sc=True, needs_layout_passes=False),
)(_sc_body)


def kernel(proprio):
    b, t, f = proprio.shape
    x = proprio.reshape(-1)
    tokens = _sc_onehot(x)
    mask = jnp.ones((b, t, f), dtype=bool)
    return tokens, mask


# speedup vs baseline: 1.0084x; 1.0084x over previous
import functools

import jax
import jax.numpy as jnp
from jax import lax
from jax.experimental import pallas as pl
from jax.experimental.pallas import tpu as pltpu
from jax.experimental.pallas import tpu_sc as plsc

EPS = 1e-06
N_BINS = 256
_B, _T, _F = 256, 20, 32
_L = 16
_NC, _NS = 2, 16
_NW = _NC * _NS
_N_ROWS = _B * _T * _F
_RPW = _N_ROWS // _NW
_BPW = _RPW // (_T * _F)
_R = 128
_TG = _R // _F
_G = _RPW // _R
_GPB = (_T * _F) // _R


def _sc_body(x_hbm, out_hbm, xin_v, buf_v, pidx_v, sem0, sem1):
    c = lax.axis_index("c")
    s = lax.axis_index("s")
    wid = s * _NC + c
    base = wid * _RPW
    pltpu.sync_copy(x_hbm.at[pl.ds(base, _RPW)], xin_v)

    iota = lax.iota(jnp.int32, _L)
    ones = jnp.full((_L,), 1.0, jnp.float32)
    zeros = jnp.zeros((_L,), jnp.float32)

    def zrow(i, carry):
        t = i // _F
        f = i % _F
        for ch in range(N_BINS // _L):
            buf_v[t, f, pl.ds(ch * _L, _L)] = zeros
        return carry

    lax.fori_loop(0, 2 * _TG * _F, zrow, 0)

    sems = (sem0, sem1)

    def outer(go, carry):
        for b in range(2):
            g = go * 2 + b
            sem = sems[b]

            @pl.when(g >= 2)
            def _wait_and_scrub():
                pltpu.make_async_copy(
                    buf_v.at[pl.ds(b * _TG, _TG)],
                    out_hbm.at[0, pl.ds(0, _TG)],
                    sem,
                ).wait()

                def scrub(j, inner):
                    r = j * _L + iota
                    cols = pidx_v[pl.ds(b * _R + j * _L, _L)]
                    plsc.store_scatter(
                        buf_v,
                        [b * _TG + (r >> 5), r & (_F - 1), cols],
                        zeros)
                    return inner

                lax.fori_loop(0, _R // _L, scrub, 0)

            def setone(j, inner):
                xv = xin_v[pl.ds(g * _R + j * _L, _L)]
                xc = jnp.clip(xv, EPS, 1.0 - EPS)
                idx = jnp.clip((xc * N_BINS).astype(jnp.int32), 0, N_BINS - 1)
                pidx_v[pl.ds(b * _R + j * _L, _L)] = idx
                r = j * _L + iota
                plsc.store_scatter(
                    buf_v,
                    [b * _TG + (r >> 5), r & (_F - 1), idx],
                    ones)
                return inner

            lax.fori_loop(0, _R // _L, setone, 0)
            bb = wid * _BPW + g // _GPB
            t0 = (g % _GPB) * _TG
            pltpu.async_copy(
                buf_v.at[pl.ds(b * _TG, _TG)],
                out_hbm.at[bb, pl.ds(t0, _TG)],
                sem,
            )
        return carry

    lax.fori_loop(0, _G // 2, outer, 0)

    for b in range(2):
        pltpu.make_async_copy(
            buf_v.at[pl.ds(b * _TG, _TG)],
            out_hbm.at[0, pl.ds(0, _TG)],
            sems[b],
        ).wait()


_sc_onehot = functools.partial(
    pl.kernel,
    out_type=jax.ShapeDtypeStruct((_B, _T, _F, N_BINS), jnp.float32),
    mesh=plsc.VectorSubcoreMesh(
        core_axis_name="c", subcore_axis_name="s",
        num_cores=_NC, num_subcores=_NS,
    ),
    scratch_types=[
        pltpu.VMEM((_RPW,), jnp.float32),
        pltpu.VMEM((2 * _TG, _F, N_BINS), jnp.float32),
        pltpu.VMEM((2 * _R,), jnp.int32),
        pltpu.SemaphoreType.DMA,
        pltpu.SemaphoreType.DMA,
    ],
    compiler_params=pltpu.CompilerParams(
        use_tc_tiling_on_sc=True, needs_layout_passes=False,
        skip_device_barrier=True),
)(_sc_body)


def kernel(proprio):
    b, t, f = proprio.shape
    x = proprio.reshape(-1)
    tokens = _sc_onehot(x)
    mask = jnp.ones((b, t, f), dtype=bool)
    return tokens, mask


# speedup vs baseline: 1.0317x; 1.0231x over previous
import functools

import jax
import jax.numpy as jnp
from jax import lax
from jax.experimental import pallas as pl
from jax.experimental.pallas import tpu as pltpu
from jax.experimental.pallas import tpu_sc as plsc

EPS = 1e-06
N_BINS = 256
_B, _T, _F = 256, 20, 32
_L = 16
_NC, _NS = 2, 16
_NW = _NC * _NS
_N_ROWS = _B * _T * _F
_RPW = _N_ROWS // _NW
_BPW = _RPW // (_T * _F)
_R = 128
_TG = _R // _F
_G = _RPW // _R
_GPB = (_T * _F) // _R


def _sc_body(x_hbm, out_hbm, xin_v, buf_v, pidx_v, sem_in, sem0, sem1):
    c = lax.axis_index("c")
    s = lax.axis_index("s")
    wid = s * _NC + c
    base = wid * _RPW
    pltpu.async_copy(x_hbm.at[pl.ds(base, _RPW)], xin_v, sem_in)

    iota = lax.iota(jnp.int32, _L)
    ones = jnp.full((_L,), 1.0, jnp.float32)
    zeros = jnp.zeros((_L,), jnp.float32)
    sems = (sem0, sem1)

    def zero_slot(b):
        def zrow(i, carry):
            t = b * _TG + i // _F
            f = i % _F
            for ch in range(N_BINS // _L):
                buf_v[t, f, pl.ds(ch * _L, _L)] = zeros
            return carry

        lax.fori_loop(0, _TG * _F, zrow, 0)

    def scrub_slot(b):
        def scrub(j, inner):
            r = j * _L + iota
            cols = pidx_v[pl.ds(b * _R + j * _L, _L)]
            plsc.store_scatter(
                buf_v, [b * _TG + (r >> 5), r & (_F - 1), cols], zeros)
            return inner

        lax.fori_loop(0, _R // _L, scrub, 0)

    def set_and_send(g, b):
        def setone(j, inner):
            xv = xin_v[pl.ds(g * _R + j * _L, _L)]
            xc = jnp.clip(xv, EPS, 1.0 - EPS)
            idx = jnp.clip((xc * N_BINS).astype(jnp.int32), 0, N_BINS - 1)
            pidx_v[pl.ds(b * _R + j * _L, _L)] = idx
            r = j * _L + iota
            plsc.store_scatter(
                buf_v, [b * _TG + (r >> 5), r & (_F - 1), idx], ones)
            return inner

        lax.fori_loop(0, _R // _L, setone, 0)
        bb = wid * _BPW + g // _GPB
        t0 = (g % _GPB) * _TG
        pltpu.async_copy(
            buf_v.at[pl.ds(b * _TG, _TG)],
            out_hbm.at[bb, pl.ds(t0, _TG)],
            sems[b],
        )

    def wait_slot(b):
        pltpu.make_async_copy(
            buf_v.at[pl.ds(b * _TG, _TG)],
            out_hbm.at[0, pl.ds(0, _TG)],
            sems[b],
        ).wait()

    zero_slot(0)
    pltpu.make_async_copy(x_hbm.at[pl.ds(0, _RPW)], xin_v, sem_in).wait()
    set_and_send(0, 0)
    zero_slot(1)
    set_and_send(1, 1)

    def outer(go, carry):
        for b in range(2):
            g = go * 2 + b
            wait_slot(b)
            scrub_slot(b)
            set_and_send(g, b)
        return carry

    lax.fori_loop(1, _G // 2, outer, 0)

    for b in range(2):
        wait_slot(b)


_sc_onehot = functools.partial(
    pl.kernel,
    out_type=jax.ShapeDtypeStruct((_B, _T, _F, N_BINS), jnp.float32),
    mesh=plsc.VectorSubcoreMesh(
        core_axis_name="c", subcore_axis_name="s",
        num_cores=_NC, num_subcores=_NS,
    ),
    scratch_types=[
        pltpu.VMEM((_RPW,), jnp.float32),
        pltpu.VMEM((2 * _TG, _F, N_BINS), jnp.float32),
        pltpu.VMEM((2 * _R,), jnp.int32),
        pltpu.SemaphoreType.DMA,
        pltpu.SemaphoreType.DMA,
        pltpu.SemaphoreType.DMA,
    ],
    compiler_params=pltpu.CompilerParams(
        use_tc_tiling_on_sc=True, needs_layout_passes=False),
)(_sc_body)


def kernel(proprio):
    b, t, f = proprio.shape
    x = proprio.reshape(-1)
    tokens = _sc_onehot(x)
    mask = jnp.ones((b, t, f), dtype=bool)
    return tokens, mask
